# baseline (speedup 1.0000x reference)
import functools

import jax
import jax.numpy as jnp
from jax import lax
from jax.experimental import pallas as pl
from jax.experimental.pallas import tpu as pltpu
from jax.experimental.pallas import tpu_sc as plsc

_EPS = 1e-5


def _sc_info():
    try:
        info = plsc.get_sparse_core_info()
        return info.num_cores, info.num_subcores
    except Exception:
        return 2, 16


def _chunks(total, step):
    out = []
    st = 0
    while st < total:
        sz = min(step, total - st)
        out.append((st, sz))
        st += sz
    return out


@functools.lru_cache(maxsize=None)
def _make_segsum(n_pad, e_pad, d, cb, nc, ns):
    nw = nc * ns
    cpw = e_pad // (nw * cb)
    nph = 2
    cpp = cpw // nph
    rps = n_pad // ns

    mesh = plsc.VectorSubcoreMesh(core_axis_name="c", subcore_axis_name="s")
    out_type = jax.ShapeDtypeStruct((nc * n_pad, d), jnp.float32)
    scratch = [
        pltpu.VMEM((cpp, cb), jnp.int32),
        pltpu.VMEM((cpp, cb), jnp.int32),
        pltpu.VMEM((cb, d), jnp.float32),
        pltpu.VMEM((cb, d), jnp.float32),
        pltpu.VMEM_SHARED((n_pad, d), jnp.float32),
        pltpu.SemaphoreType.DMA,
        pltpu.SemaphoreType.DMA,
        pltpu.SemaphoreType.DMA,
        pltpu.SemaphoreType.DMA,
    ]

    def body(x_hbm, srcs_hbm, dsts_hbm, out_p, src_slab, dst_slab,
             rows0, rows1, acc, gsem0, gsem1, ssem0, ssem1):
        c = lax.axis_index("c")
        s = lax.axis_index("s")
        wid = s * nc + c
        bufs = ((rows0, gsem0, ssem0), (rows1, gsem1, ssem1))

        def zr(i, carry):
            def zc(j, carry2):
                rows0[i, pl.ds(j * 16, 16)] = jnp.zeros((16,), jnp.float32)
                return carry2
            return lax.fori_loop(0, d // 16, zc, carry)
        lax.fori_loop(0, cb, zr, 0)
        for (st, sz) in _chunks(rps, cb):
            pltpu.sync_copy(rows0.at[pl.ds(0, sz)], acc.at[pl.ds(s * rps + st, sz)])
        plsc.subcore_barrier()

        for ph in range(nph):
            base = wid * cpw + ph * cpp
            pltpu.sync_copy(srcs_hbm.at[pl.ds(base, cpp)], src_slab)
            pltpu.sync_copy(dsts_hbm.at[pl.ds(base, cpp)], dst_slab)
            pltpu.async_copy(x_hbm.at[src_slab.at[0]], rows0, gsem0)

            def step(o, carry):
                for b, (rb, gsem, ssem) in enumerate(bufs):
                    g = o * 2 + b
                    orb, ogsem, ossem = bufs[1 - b]

                    pltpu.make_async_copy(x_hbm.at[src_slab.at[g]], rb, gsem).wait()
                    pltpu.async_copy(rb, acc.at[dst_slab.at[g]], ssem, add=True)

                    @pl.when(g >= 1)
                    def _():
                        pltpu.make_async_copy(
                            orb, acc.at[dst_slab.at[g - 1]], ossem).wait()

                    @pl.when(g + 1 < cpp)
                    def _():
                        pltpu.async_copy(x_hbm.at[src_slab.at[g + 1]], orb, ogsem)
                return carry
            lax.fori_loop(0, cpp // 2, step, 0)
            pltpu.make_async_copy(
                rows1, acc.at[dst_slab.at[cpp - 1]], ssem1).wait()
        plsc.subcore_barrier()

        for (st, sz) in _chunks(rps, cb):
            pltpu.sync_copy(acc.at[pl.ds(s * rps + st, sz)],
                            out_p.at[pl.ds(c * n_pad + s * rps + st, sz)])

    return pl.kernel(body, mesh=mesh, out_type=out_type, scratch_types=scratch)


@functools.lru_cache(maxsize=None)
def _make_deg(n_pad, e_pad, cb, nc, ns, d):
    nw = nc * ns
    cpw = e_pad // (nw * cb)
    rps = n_pad // ns

    mesh = plsc.VectorSubcoreMesh(core_axis_name="c", subcore_axis_name="s")
    out_type = jax.ShapeDtypeStruct((nc * n_pad, d), jnp.float32)
    scratch = [
        pltpu.VMEM((cpw, cb), jnp.int32),
        pltpu.VMEM((cb, d), jnp.float32),
        pltpu.VMEM_SHARED((n_pad, d), jnp.float32),
    ]

    def body(dsts_hbm, out_d, dst_slab, ones, dacc):
        c = lax.axis_index("c")
        s = lax.axis_index("s")
        wid = s * nc + c

        pltpu.sync_copy(dsts_hbm.at[pl.ds(wid * cpw, cpw)], dst_slab)

        def zo(i, carry):
            def zc(j, carry2):
                ones[i, pl.ds(j * 16, 16)] = jnp.zeros((16,), jnp.float32)
                return carry2
            return lax.fori_loop(0, d // 16, zc, carry)
        lax.fori_loop(0, cb, zo, 0)
        for (st, sz) in _chunks(rps, cb):
            pltpu.sync_copy(ones.at[pl.ds(0, sz)], dacc.at[pl.ds(s * rps + st, sz)])
        def so(i, carry):
            def sc(j, carry2):
                ones[i, pl.ds(j * 16, 16)] = jnp.ones((16,), jnp.float32)
                return carry2
            return lax.fori_loop(0, d // 16, sc, carry)
        lax.fori_loop(0, cb, so, 0)
        plsc.subcore_barrier()

        def step(j, carry):
            pltpu.sync_copy(ones, dacc.at[dst_slab.at[j]], add=True)
            return carry
        lax.fori_loop(0, cpw, step, 0)
        plsc.subcore_barrier()

        for (st, sz) in _chunks(rps, cb):
            pltpu.sync_copy(dacc.at[pl.ds(s * rps + st, sz)],
                            out_d.at[pl.ds(c * n_pad + s * rps + st, sz)])

    return pl.kernel(body, mesh=mesh, out_type=out_type, scratch_types=scratch)


@functools.lru_cache(maxsize=None)
def _make_dense(n_nodes, n_pad, d, with_bn, out_pre):
    def body(*refs):
        if with_bn:
            p_ref, dp_ref, x_ref, wc, bc, ws, bs, g, b = refs[:9]
            outs = refs[9:]
        else:
            p_ref, dp_ref, x_ref, wc, bc, ws, bs = refs[:7]
            outs = refs[7:]
        p = p_ref[...]
        agg = p[0:n_nodes] + p[n_pad:n_pad + n_nodes]
        dp = dp_ref[...]
        deg = dp[0:n_nodes, 0:1] + dp[n_pad:n_pad + n_nodes, 0:1]
        a = agg / jnp.maximum(deg, 1.0)
        x = x_ref[...]
        y = (jnp.dot(a, wc[...], preferred_element_type=jnp.float32)
             + bc[...][None, :]
             + jnp.dot(x, ws[...], preferred_element_type=jnp.float32)
             + bs[...][None, :])
        k = 0
        if out_pre:
            outs[k][...] = y
            k += 1
        if with_bn:
            m = jnp.mean(y, axis=0, keepdims=True)
            v = jnp.mean((y - m) ** 2, axis=0, keepdims=True)
            yn = g[...][None, :] * (y - m) / jnp.sqrt(v + _EPS) + b[...][None, :]
            outs[k][...] = jnp.maximum(yn, 0.0)
        else:
            outs[k][...] = y

    n_out = (1 if out_pre else 0) + 1
    return pl.pallas_call(
        body,
        out_shape=[jax.ShapeDtypeStruct((n_nodes, d), jnp.float32)] * n_out,
    )


def kernel(x, edge_index, W_conv0, b_conv0, W_skip0, b_skip0, W_conv1, b_conv1,
           W_skip1, b_skip1, W_conv2, b_conv2, W_skip2, b_skip2, gamma0, beta0,
           gamma1, beta1):
    n, d = x.shape
    e = edge_index.shape[1]
    nc, ns = _sc_info()
    nw = nc * ns
    cb = 128
    e_pad = -(-e // (nw * cb * 8)) * (nw * cb * 8)
    n_pad = -(-(n + 16) // (ns * 8)) * (ns * 8)

    src = edge_index[0]
    dst = edge_index[1]
    if e_pad > e:
        fill = jnp.arange(e_pad - e, dtype=jnp.int32)
        src = jnp.concatenate([src, fill % 16])
        dst = jnp.concatenate([dst, n + (fill % (n_pad - n))])
    srcs2 = src.reshape(e_pad // cb, cb)
    dsts2 = dst.reshape(e_pad // cb, cb)

    seg = _make_segsum(n_pad, e_pad, d, cb, nc, ns)
    deg = _make_deg(n_pad, e_pad, cb, nc, ns, d)
    dense_bn = _make_dense(n, n_pad, d, True, False)
    dense_bn_pre = _make_dense(n, n_pad, d, True, True)
    dense_last = _make_dense(n, n_pad, d, False, False)

    dp = deg(dsts2)
    p0 = seg(x, srcs2, dsts2)
    (x1,) = dense_bn(p0, dp, x, W_conv0, b_conv0, W_skip0, b_skip0, gamma0, beta0)
    p1 = seg(x1, srcs2, dsts2)
    h, x2 = dense_bn_pre(p1, dp, x1, W_conv1, b_conv1, W_skip1, b_skip1, gamma1, beta1)
    p2 = seg(x2, srcs2, dsts2)
    (y,) = dense_last(p2, dp, x2, W_conv2, b_conv2, W_skip2, b_skip2)
    return (h, y)


# speedup vs baseline: 1.1486x; 1.1486x over previous
import functools

import jax
import jax.numpy as jnp
from jax import lax
from jax.experimental import pallas as pl
from jax.experimental.pallas import tpu as pltpu
from jax.experimental.pallas import tpu_sc as plsc

_EPS = 1e-5


def _sc_info():
    try:
        info = plsc.get_sparse_core_info()
        return info.num_cores, info.num_subcores
    except Exception:
        return 2, 16


def _chunks(total, step):
    out = []
    st = 0
    while st < total:
        sz = min(step, total - st)
        out.append((st, sz))
        st += sz
    return out


@functools.lru_cache(maxsize=None)
def _make_segsum(n_pad, e_pad, d, cb, nc, ns, with_deg=False):
    nw = nc * ns
    cpw = e_pad // (nw * cb)
    nph = 2
    cpp = cpw // nph
    rps = n_pad // ns

    mesh = plsc.VectorSubcoreMesh(core_axis_name="c", subcore_axis_name="s")
    out_type = [jax.ShapeDtypeStruct((nc * n_pad, d), jnp.float32)]
    if with_deg:
        out_type.append(jax.ShapeDtypeStruct((nc * n_pad, d), jnp.float32))
    scratch = [
        pltpu.VMEM((cpp, cb), jnp.int32),
        pltpu.VMEM((cpp, cb), jnp.int32),
        pltpu.VMEM((cb, d), jnp.float32),
        pltpu.VMEM((cb, d), jnp.float32),
        pltpu.VMEM_SHARED((n_pad, d), jnp.float32),
        pltpu.SemaphoreType.DMA,
        pltpu.SemaphoreType.DMA,
        pltpu.SemaphoreType.DMA,
        pltpu.SemaphoreType.DMA,
    ]

    def body(x_hbm, srcs_hbm, dsts_hbm, *rest):
        if with_deg:
            (out_p, out_d, src_slab, dst_slab, rows0, rows1, acc,
             gsem0, gsem1, ssem0, ssem1) = rest
        else:
            (out_p, src_slab, dst_slab, rows0, rows1, acc,
             gsem0, gsem1, ssem0, ssem1) = rest
        c = lax.axis_index("c")
        s = lax.axis_index("s")
        wid = s * nc + c
        bufs = ((rows0, gsem0, ssem0), (rows1, gsem1, ssem1))

        def fill(val):
            def zr(i, carry):
                def zc(j, carry2):
                    rows0[i, pl.ds(j * 16, 16)] = jnp.full((16,), val, jnp.float32)
                    return carry2
                return lax.fori_loop(0, d // 16, zc, carry)
            lax.fori_loop(0, cb, zr, 0)

        def zero_acc():
            for (st, sz) in _chunks(rps, cb):
                pltpu.async_copy(rows0.at[pl.ds(0, sz)],
                                 acc.at[pl.ds(s * rps + st, sz)], ssem0)
            for (st, sz) in _chunks(rps, cb):
                pltpu.make_async_copy(rows0.at[pl.ds(0, sz)],
                                      acc.at[pl.ds(s * rps + st, sz)], ssem0).wait()

        def writeout(dst_hbm):
            for (st, sz) in _chunks(rps, cb):
                pltpu.async_copy(acc.at[pl.ds(s * rps + st, sz)],
                                 dst_hbm.at[pl.ds(c * n_pad + s * rps + st, sz)],
                                 ssem1)
            for (st, sz) in _chunks(rps, cb):
                pltpu.make_async_copy(acc.at[pl.ds(s * rps + st, sz)],
                                      dst_hbm.at[pl.ds(c * n_pad + s * rps + st, sz)],
                                      ssem1).wait()

        fill(0.0)
        zero_acc()
        plsc.subcore_barrier()

        if with_deg:
            fill(1.0)
            for ph in range(nph):
                base = wid * cpw + ph * cpp
                pltpu.sync_copy(dsts_hbm.at[pl.ds(base, cpp)], dst_slab)

                def dstep(o, carry):
                    for b in (0, 1):
                        g = o * 2 + b
                        sem = (ssem0, ssem1)[b]
                        osem = (ssem0, ssem1)[1 - b]
                        pltpu.async_copy(rows0, acc.at[dst_slab.at[g]], sem,
                                         add=True)

                        @pl.when(g >= 1)
                        def _():
                            pltpu.make_async_copy(
                                rows0, acc.at[dst_slab.at[g - 1]], osem).wait()
                    return carry
                lax.fori_loop(0, cpp // 2, dstep, 0)
                pltpu.make_async_copy(
                    rows0, acc.at[dst_slab.at[cpp - 1]], ssem1).wait()
            plsc.subcore_barrier()
            writeout(out_d)
            plsc.subcore_barrier()
            fill(0.0)
            zero_acc()
            plsc.subcore_barrier()

        for ph in range(nph):
            base = wid * cpw + ph * cpp
            pltpu.sync_copy(srcs_hbm.at[pl.ds(base, cpp)], src_slab)
            pltpu.sync_copy(dsts_hbm.at[pl.ds(base, cpp)], dst_slab)
            pltpu.async_copy(x_hbm.at[src_slab.at[0]], rows0, gsem0)

            def step(o, carry):
                for b, (rb, gsem, ssem) in enumerate(bufs):
                    g = o * 2 + b
                    orb, ogsem, ossem = bufs[1 - b]

                    @pl.when(g + 1 < cpp)
                    def _():
                        pltpu.async_copy(x_hbm.at[src_slab.at[g + 1]], orb, ogsem)

                    pltpu.make_async_copy(x_hbm.at[src_slab.at[g]], rb, gsem).wait()
                    pltpu.sync_copy(rb, acc.at[dst_slab.at[g]], add=True)
                return carry
            lax.fori_loop(0, cpp // 2, step, 0)
        plsc.subcore_barrier()

        writeout(out_p)

    return pl.kernel(body, mesh=mesh, out_type=out_type, scratch_types=scratch)


@functools.lru_cache(maxsize=None)
def _make_dense(n_nodes, n_pad, d, with_bn, out_pre):
    def body(*refs):
        if with_bn:
            p_ref, dp_ref, x_ref, wc, bc, ws, bs, g, b = refs[:9]
            outs = refs[9:]
        else:
            p_ref, dp_ref, x_ref, wc, bc, ws, bs = refs[:7]
            outs = refs[7:]
        p = p_ref[...]
        agg = p[0:n_nodes] + p[n_pad:n_pad + n_nodes]
        dp = dp_ref[...]
        deg = dp[0:n_nodes, 0:1] + dp[n_pad:n_pad + n_nodes, 0:1]
        a = agg / jnp.maximum(deg, 1.0)
        x = x_ref[...]
        y = (jnp.dot(a, wc[...], preferred_element_type=jnp.float32)
             + bc[...][None, :]
             + jnp.dot(x, ws[...], preferred_element_type=jnp.float32)
             + bs[...][None, :])
        k = 0
        if out_pre:
            outs[k][...] = y
            k += 1
        if with_bn:
            m = jnp.mean(y, axis=0, keepdims=True)
            v = jnp.mean((y - m) ** 2, axis=0, keepdims=True)
            yn = g[...][None, :] * (y - m) / jnp.sqrt(v + _EPS) + b[...][None, :]
            outs[k][...] = jnp.maximum(yn, 0.0)
        else:
            outs[k][...] = y

    n_out = (1 if out_pre else 0) + 1
    return pl.pallas_call(
        body,
        out_shape=[jax.ShapeDtypeStruct((n_nodes, d), jnp.float32)] * n_out,
    )


def kernel(x, edge_index, W_conv0, b_conv0, W_skip0, b_skip0, W_conv1, b_conv1,
           W_skip1, b_skip1, W_conv2, b_conv2, W_skip2, b_skip2, gamma0, beta0,
           gamma1, beta1):
    n, d = x.shape
    e = edge_index.shape[1]
    nc, ns = _sc_info()
    nw = nc * ns
    cb = 128
    e_pad = -(-e // (nw * cb * 8)) * (nw * cb * 8)
    n_pad = -(-(n + 16) // (ns * 8)) * (ns * 8)

    src = edge_index[0]
    dst = edge_index[1]
    if e_pad > e:
        fill = jnp.arange(e_pad - e, dtype=jnp.int32)
        src = jnp.concatenate([src, fill % 16])
        dst = jnp.concatenate([dst, n + (fill % (n_pad - n))])
    srcs2 = src.reshape(e_pad // cb, cb)
    dsts2 = dst.reshape(e_pad // cb, cb)

    seg = _make_segsum(n_pad, e_pad, d, cb, nc, ns)
    seg_deg = _make_segsum(n_pad, e_pad, d, cb, nc, ns, with_deg=True)
    dense_bn = _make_dense(n, n_pad, d, True, False)
    dense_bn_pre = _make_dense(n, n_pad, d, True, True)
    dense_last = _make_dense(n, n_pad, d, False, False)

    p0, dp = seg_deg(x, srcs2, dsts2)
    (x1,) = dense_bn(p0, dp, x, W_conv0, b_conv0, W_skip0, b_skip0, gamma0, beta0)
    (p1,) = seg(x1, srcs2, dsts2)
    h, x2 = dense_bn_pre(p1, dp, x1, W_conv1, b_conv1, W_skip1, b_skip1, gamma1, beta1)
    (p2,) = seg(x2, srcs2, dsts2)
    (y,) = dense_last(p2, dp, x2, W_conv2, b_conv2, W_skip2, b_skip2)
    return (h, y)


# speedup vs baseline: 1.1733x; 1.0215x over previous
import functools

import jax
import jax.numpy as jnp
from jax import lax
from jax.experimental import pallas as pl
from jax.experimental.pallas import tpu as pltpu
from jax.experimental.pallas import tpu_sc as plsc

_EPS = 1e-5


def _sc_info():
    try:
        info = plsc.get_sparse_core_info()
        return info.num_cores, info.num_subcores
    except Exception:
        return 2, 16


def _chunks(total, step):
    out = []
    st = 0
    while st < total:
        sz = min(step, total - st)
        out.append((st, sz))
        st += sz
    return out


@functools.lru_cache(maxsize=None)
def _make_segsum(n_pad, e_pad, d, cb, nc, ns, with_deg=False):
    nw = nc * ns
    cpw = e_pad // (nw * cb)
    nph = 2
    cpp = cpw // nph
    rps = n_pad // ns

    mesh = plsc.VectorSubcoreMesh(core_axis_name="c", subcore_axis_name="s")
    out_type = [jax.ShapeDtypeStruct((nc * n_pad, d), jnp.float32)]
    if with_deg:
        out_type.append(jax.ShapeDtypeStruct((nc * n_pad, d), jnp.float32))
    scratch = [
        pltpu.VMEM((cpp, cb), jnp.int32),
        pltpu.VMEM((cpp, cb), jnp.int32),
        pltpu.VMEM((cb, d), jnp.float32),
        pltpu.VMEM((cb, d), jnp.float32),
        pltpu.VMEM_SHARED((n_pad, d), jnp.float32),
        pltpu.SemaphoreType.DMA,
        pltpu.SemaphoreType.DMA,
        pltpu.SemaphoreType.DMA,
        pltpu.SemaphoreType.DMA,
    ]

    def body(x_hbm, srcs_hbm, dsts_hbm, *rest):
        if with_deg:
            (out_p, out_d, src_slab, dst_slab, rows0, rows1, acc,
             gsem0, gsem1, ssem0, ssem1) = rest
        else:
            (out_p, src_slab, dst_slab, rows0, rows1, acc,
             gsem0, gsem1, ssem0, ssem1) = rest
        c = lax.axis_index("c")
        s = lax.axis_index("s")
        wid = s * nc + c
        bufs = ((rows0, gsem0, ssem0), (rows1, gsem1, ssem1))

        def fill(val):
            def zr(i, carry):
                def zc(j, carry2):
                    rows0[i, pl.ds(j * 16, 16)] = jnp.full((16,), val, jnp.float32)
                    return carry2
                return lax.fori_loop(0, d // 16, zc, carry)
            lax.fori_loop(0, cb, zr, 0)

        def zero_acc():
            for (st, sz) in _chunks(rps, cb):
                pltpu.async_copy(rows0.at[pl.ds(0, sz)],
                                 acc.at[pl.ds(s * rps + st, sz)], ssem0)
            for (st, sz) in _chunks(rps, cb):
                pltpu.make_async_copy(rows0.at[pl.ds(0, sz)],
                                      acc.at[pl.ds(s * rps + st, sz)], ssem0).wait()

        def writeout(dst_hbm):
            for (st, sz) in _chunks(rps, cb):
                pltpu.async_copy(acc.at[pl.ds(s * rps + st, sz)],
                                 dst_hbm.at[pl.ds(c * n_pad + s * rps + st, sz)],
                                 ssem1)
            for (st, sz) in _chunks(rps, cb):
                pltpu.make_async_copy(acc.at[pl.ds(s * rps + st, sz)],
                                      dst_hbm.at[pl.ds(c * n_pad + s * rps + st, sz)],
                                      ssem1).wait()

        fill(0.0)
        zero_acc()
        plsc.subcore_barrier()

        if with_deg:
            fill(1.0)
            for ph in range(nph):
                base = wid * cpw + ph * cpp
                pltpu.sync_copy(dsts_hbm.at[pl.ds(base, cpp)], dst_slab)

                def dstep(o, carry):
                    for b in (0, 1):
                        g = o * 2 + b
                        sem = (ssem0, ssem1)[b]
                        osem = (ssem0, ssem1)[1 - b]
                        pltpu.async_copy(rows0, acc.at[dst_slab.at[g]], sem,
                                         add=True)

                        @pl.when(g >= 1)
                        def _():
                            pltpu.make_async_copy(
                                rows0, acc.at[dst_slab.at[g - 1]], osem).wait()
                    return carry
                lax.fori_loop(0, cpp // 2, dstep, 0)
                pltpu.make_async_copy(
                    rows0, acc.at[dst_slab.at[cpp - 1]], ssem1).wait()
            plsc.subcore_barrier()
            writeout(out_d)
            plsc.subcore_barrier()
            fill(0.0)
            zero_acc()
            plsc.subcore_barrier()

        for ph in range(nph):
            base = wid * cpw + ph * cpp
            pltpu.sync_copy(srcs_hbm.at[pl.ds(base, cpp)], src_slab)
            pltpu.sync_copy(dsts_hbm.at[pl.ds(base, cpp)], dst_slab)
            pltpu.async_copy(x_hbm.at[src_slab.at[0]], rows0, gsem0)

            def step(o, carry):
                for b, (rb, gsem, ssem) in enumerate(bufs):
                    g = o * 2 + b
                    orb, ogsem, ossem = bufs[1 - b]

                    @pl.when(g + 1 < cpp)
                    def _():
                        pltpu.async_copy(x_hbm.at[src_slab.at[g + 1]], orb, ogsem)

                    pltpu.make_async_copy(x_hbm.at[src_slab.at[g]], rb, gsem).wait()
                    pltpu.sync_copy(rb, acc.at[dst_slab.at[g]], add=True)
                return carry
            lax.fori_loop(0, cpp // 2, step, 0)
        plsc.subcore_barrier()

        writeout(out_p)

    return pl.kernel(body, mesh=mesh, out_type=out_type, scratch_types=scratch)


@functools.lru_cache(maxsize=None)
def _make_dense(n_nodes, n_pad, d, with_bn, out_pre, deg_wide=False):
    def body(*refs):
        if with_bn:
            p_ref, dp_ref, x_ref, wc, bc, ws, bs, g, b = refs[:9]
            outs = refs[9:]
        else:
            p_ref, dp_ref, x_ref, wc, bc, ws, bs = refs[:7]
            outs = refs[7:]
        p = p_ref[...]
        agg = p[0:n_nodes] + p[n_pad:n_pad + n_nodes]
        if deg_wide:
            dpv = dp_ref[...]
            deg = dpv[0:n_nodes, 0:1] + dpv[n_pad:n_pad + n_nodes, 0:1]
            invd = 1.0 / jnp.maximum(deg, 1.0)
            outs[0][...] = invd
            outs = outs[1:]
        else:
            invd = dp_ref[...]
        a = agg * invd
        x = x_ref[...]
        y = (jnp.dot(a, wc[...], preferred_element_type=jnp.float32)
             + bc[...][None, :]
             + jnp.dot(x, ws[...], preferred_element_type=jnp.float32)
             + bs[...][None, :])
        k = 0
        if out_pre:
            outs[k][...] = y
            k += 1
        if with_bn:
            m = jnp.mean(y, axis=0, keepdims=True)
            v = jnp.mean((y - m) ** 2, axis=0, keepdims=True)
            yn = g[...][None, :] * (y - m) / jnp.sqrt(v + _EPS) + b[...][None, :]
            outs[k][...] = jnp.maximum(yn, 0.0)
        else:
            outs[k][...] = y

    out_shape = ([jax.ShapeDtypeStruct((n_nodes, 1), jnp.float32)] if deg_wide else [])
    if out_pre:
        out_shape.append(jax.ShapeDtypeStruct((n_nodes, d), jnp.float32))
    out_shape.append(jax.ShapeDtypeStruct((n_nodes, d), jnp.float32))
    return pl.pallas_call(body, out_shape=out_shape)


def kernel(x, edge_index, W_conv0, b_conv0, W_skip0, b_skip0, W_conv1, b_conv1,
           W_skip1, b_skip1, W_conv2, b_conv2, W_skip2, b_skip2, gamma0, beta0,
           gamma1, beta1):
    n, d = x.shape
    e = edge_index.shape[1]
    nc, ns = _sc_info()
    nw = nc * ns
    cb = 128
    e_pad = -(-e // (nw * cb * 8)) * (nw * cb * 8)
    n_pad = -(-(n + 16) // (ns * 8)) * (ns * 8)

    src = edge_index[0]
    dst = edge_index[1]
    if e_pad > e:
        fill = jnp.arange(e_pad - e, dtype=jnp.int32)
        src = jnp.concatenate([src, fill % 16])
        dst = jnp.concatenate([dst, n + (fill % (n_pad - n))])
    srcs2 = src.reshape(e_pad // cb, cb)
    dsts2 = dst.reshape(e_pad // cb, cb)

    seg = _make_segsum(n_pad, e_pad, d, cb, nc, ns)
    seg_deg = _make_segsum(n_pad, e_pad, d, cb, nc, ns, with_deg=True)
    dense_bn = _make_dense(n, n_pad, d, True, False, deg_wide=True)
    dense_bn_pre = _make_dense(n, n_pad, d, True, True)
    dense_last = _make_dense(n, n_pad, d, False, False)

    p0, dp = seg_deg(x, srcs2, dsts2)
    invd, x1 = dense_bn(p0, dp, x, W_conv0, b_conv0, W_skip0, b_skip0, gamma0, beta0)
    (p1,) = seg(x1, srcs2, dsts2)
    h, x2 = dense_bn_pre(p1, invd, x1, W_conv1, b_conv1, W_skip1, b_skip1, gamma1, beta1)
    (p2,) = seg(x2, srcs2, dsts2)
    (y,) = dense_last(p2, invd, x2, W_conv2, b_conv2, W_skip2, b_skip2)
    return (h, y)
